# baseline (speedup 1.0000x reference)
import jax
import jax.numpy as jnp
from jax import lax
from jax.experimental import pallas as pl
from jax.experimental.pallas import tpu as pltpu

BETA = 0.25
ROW_BLOCK = 3136


def _vq_body(x, e_ref, et_ref, q_ref, part_ref):
    e = e_ref[:]
    e2 = jnp.sum(e * e, axis=0, keepdims=True)
    score = 2.0 * jnp.dot(x, e, preferred_element_type=jnp.float32) - e2
    ind = jnp.argmax(score, axis=1)
    onehot = (lax.broadcasted_iota(jnp.int32, score.shape, 1)
              == ind[:, None]).astype(jnp.float32)
    q = jnp.dot(onehot, et_ref[:], preferred_element_type=jnp.float32)
    q_ref[:] = q
    d = q - x
    part_ref[:] = jnp.full((1, 1, 128), jnp.sum(d * d), jnp.float32)


def _vq_block1(xa_ref, wa_ref, bb_ref, e_ref, et_ref, q_ref, part_ref):
    x = jnp.dot(xa_ref[:], wa_ref[:], preferred_element_type=jnp.float32) + bb_ref[:]
    _vq_body(x, e_ref, et_ref, q_ref, part_ref)


def _vq_block2(xa_ref, xb_ref, wa_ref, wb_ref, bb_ref, e_ref, et_ref,
               q_ref, part_ref):
    x = (jnp.dot(xa_ref[:], wa_ref[:], preferred_element_type=jnp.float32)
         + jnp.dot(xb_ref[:], wb_ref[:], preferred_element_type=jnp.float32)
         + bb_ref[:])
    _vq_body(x, e_ref, et_ref, q_ref, part_ref)


def _quantize(parts_in, pre_w, pre_b, embed, out_hw):
    d = embed.shape[0]
    k = embed.shape[1]
    flats = [p.reshape(-1, p.shape[3]) for p in parts_in]
    n = flats[0].shape[0]
    grid = n // ROW_BLOCK
    ws = []
    c0 = 0
    for p in parts_in:
        ci = p.shape[3]
        ws.append(pre_w[:, c0:c0 + ci, 0, 0].T)
        c0 += ci
    rspec = lambda c: pl.BlockSpec((ROW_BLOCK, c), lambda i: (i, 0))
    wspec = lambda s: pl.BlockSpec(s, lambda i: (0,) * len(s))
    body = _vq_block1 if len(parts_in) == 1 else _vq_block2
    q, parts = pl.pallas_call(
        body,
        grid=(grid,),
        in_specs=[rspec(p.shape[3]) for p in parts_in]
        + [wspec(w.shape) for w in ws]
        + [wspec((1, d)), wspec((d, k)), wspec((k, d))],
        out_specs=[
            rspec(d),
            pl.BlockSpec((1, 1, 128), lambda i: (i, 0, 0)),
        ],
        out_shape=[
            jax.ShapeDtypeStruct((n, d), jnp.float32),
            jax.ShapeDtypeStruct((grid, 1, 128), jnp.float32),
        ],
    )(*flats, *ws, pre_b[None, :], embed, embed.T)
    diff = jnp.sum(parts[:, 0, 0]) / (n * d)
    b = parts_in[0].shape[0]
    return q.reshape(b, out_hw, out_hw, d), diff


def _conv(x, w, b, stride=1, pad=0):
    y = lax.conv_general_dilated(x, w.transpose(2, 3, 1, 0), (stride, stride),
                                 [(pad, pad), (pad, pad)],
                                 dimension_numbers=('NHWC', 'HWIO', 'NHWC'),
                                 preferred_element_type=jnp.float32)
    return y + b[None, None, None, :]


def _conv_t(x, w, b, stride=2, pad=1):
    wt = jnp.flip(w, (2, 3)).transpose(2, 3, 0, 1)
    kk = w.shape[2]
    p = kk - 1 - pad
    y = lax.conv_general_dilated(x, wt, (1, 1), [(p, p), (p, p)],
                                 lhs_dilation=(stride, stride),
                                 dimension_numbers=('NHWC', 'HWIO', 'NHWC'),
                                 preferred_element_type=jnp.float32)
    return y + b[None, None, None, :]


def _res_block(x, p):
    o = jax.nn.relu(x)
    o = _conv(o, p['w1'], p['b1'], 1, 1)
    o = jax.nn.relu(o)
    o = _conv(o, p['w2'], p['b2'], 1, 0)
    return o + x


def _encoder_s4(x, p):
    x = jax.nn.relu(_conv(x, p['w0'], p['b0'], 2, 1))
    x = jax.nn.relu(_conv(x, p['w1'], p['b1'], 2, 1))
    x = _conv(x, p['w2'], p['b2'], 1, 1)
    for rp in p['res']:
        x = _res_block(x, rp)
    return jax.nn.relu(x)


def _encoder_s2(x, p):
    x = jax.nn.relu(_conv(x, p['w0'], p['b0'], 2, 1))
    x = _conv(x, p['w1'], p['b1'], 1, 1)
    for rp in p['res']:
        x = _res_block(x, rp)
    return jax.nn.relu(x)


def _decoder_s2(x, p):
    x = _conv(x, p['w0'], p['b0'], 1, 1)
    for rp in p['res']:
        x = _res_block(x, rp)
    x = jax.nn.relu(x)
    return _conv_t(x, p['ct_w'], p['ct_b'], 2, 1)


def _dec_stage1(xa_ref, xb_ref, w0_ref, b0_ref,
                r1w1_ref, r1b1_ref, r1w2_ref, r1b2_ref,
                r2w1_ref, r2b1_ref, r2w2_ref, r2b2_ref,
                ctw_ref, ctb_ref, ct2w_ref, ct2b_ref,
                z_ref, s0, s1, s2, sc0, sc1, y_s):
    f32 = jnp.float32
    H = 56
    M = H * H
    S = (s0, s1, s2)
    SC = (sc0, sc1)

    @pl.when(pl.program_id(0) == 0)
    def _zero():
        s0[:] = jnp.zeros_like(s0)
        s1[:] = jnp.zeros_like(s1)
        s2[:] = jnp.zeros_like(s2)
        sc0[:] = jnp.zeros_like(sc0)
        sc1[:] = jnp.zeros_like(sc1)

    def put(a):
        s0[1:57, 1:56, :] = a[:, 0:55, :]
        s1[1:57, 0:56, :] = a
        s2[1:57, 0:55, :] = a[:, 1:56, :]

    def conv9(w_ref, bias):
        acc = jnp.broadcast_to(bias, (M, w_ref.shape[2]))
        for dy in range(3):
            for dx in range(3):
                lhs = S[dx][dy:dy + H, :, :].reshape(M, 128)
                acc = acc + jnp.dot(lhs, w_ref[dy * 3 + dx],
                                    preferred_element_type=f32)
        return acc

    put(jnp.concatenate([xa_ref[0], xb_ref[0]], axis=2))

    y = conv9(w0_ref, b0_ref[:])
    y_s[:] = y
    put(jax.nn.relu(y).reshape(H, H, 128))

    for (w1_ref, b1_ref, w2_ref, b2_ref) in (
            (r1w1_ref, r1b1_ref, r1w2_ref, r1b2_ref),
            (r2w1_ref, r2b1_ref, r2w2_ref, r2b2_ref)):
        o = conv9(w1_ref, b1_ref[:])
        o = jax.nn.relu(o)
        o = jnp.dot(o, w2_ref[:], preferred_element_type=f32) + b2_ref[:]
        y = o + y_s[:]
        y_s[:] = y
        put(jax.nn.relu(y).reshape(H, H, 128))

    for r in range(2):
        for c in range(2):
            rc = r * 2 + c
            acc = jnp.broadcast_to(ctb_ref[:], (M, 64))
            for a in range(2):
                for b in range(2):
                    lhs = S[b + c][a + r:a + r + H, :, :].reshape(M, 128)
                    acc = acc + jnp.dot(lhs, ctw_ref[rc * 4 + a * 2 + b],
                                        preferred_element_type=f32)
            av = jax.nn.relu(acc).reshape(H, H, 64)
            if c == 0:
                SC[r][1:57, 0:56, 64:128] = av
                SC[r][1:57, 0:55, 192:256] = av[:, 1:56, :]
            else:
                SC[r][1:57, 1:56, 0:64] = av[:, 0:55, :]
                SC[r][1:57, 0:56, 128:192] = av

    out = jnp.broadcast_to(ct2b_ref[:], (M, 48))
    for g, (s, r1) in enumerate(((-1, 1), (0, 0), (0, 1), (1, 0))):
        lhs = SC[r1][1 + s:57 + s, :, :].reshape(M, 256)
        out = out + jnp.dot(lhs, ct2w_ref[g], preferred_element_type=f32)
    z_ref[0, :, :, :] = out.reshape(H, H, 48)


def _phase_w_t(w):
    ci, co = w.shape[0], w.shape[1]
    wt = jnp.flip(w, (2, 3)).transpose(2, 3, 0, 1)
    return (wt.reshape(2, 2, 2, 2, ci, co)
              .transpose(1, 3, 0, 2, 4, 5).reshape(16, ci, co))


def _ct2_group_w(w):
    wp = _phase_w_t(w)
    r6 = jnp.zeros((4, 4, 64, 4, 4, 3), jnp.float32)
    for u1 in range(2):
        for r2 in range(2):
            for a in range(2):
                gr = u1 + r2 + a
                pr = 2 * u1 + r2
                for v1 in range(2):
                    for c2 in range(2):
                        for b in range(2):
                            gc = v1 + c2 + b
                            pc = 2 * v1 + c2
                            wtap = wp[(r2 * 2 + c2) * 4 + a * 2 + b]
                            r6 = r6.at[gr, gc, :, pr, pc, :].set(wtap)
    return r6.reshape(4, 256, 48)


def _decoder_s4_pallas(xa, xb, p):
    n = xa.shape[0]
    f32 = jnp.float32
    w0 = p['w0'].transpose(2, 3, 1, 0).reshape(9, 128, 128)
    r = p['res']
    rw1 = [rp['w1'].transpose(2, 3, 1, 0).reshape(9, 128, 32) for rp in r]
    rw2 = [rp['w2'][:, :, 0, 0].T for rp in r]
    ctw = _phase_w_t(p['ct1_w'])
    ct2w = _ct2_group_w(p['ct2_w'])
    ct2b = jnp.tile(p['ct2_b'], 16)
    wspec = lambda s: pl.BlockSpec(s, lambda i: (0,) * len(s))
    z2 = pl.pallas_call(
        _dec_stage1,
        grid=(n,),
        in_specs=[
            pl.BlockSpec((1, 56, 56, 64), lambda i: (i, 0, 0, 0)),
            pl.BlockSpec((1, 56, 56, 64), lambda i: (i, 0, 0, 0)),
            wspec((9, 128, 128)), wspec((1, 128)),
            wspec((9, 128, 32)), wspec((1, 32)), wspec((32, 128)), wspec((1, 128)),
            wspec((9, 128, 32)), wspec((1, 32)), wspec((32, 128)), wspec((1, 128)),
            wspec((16, 128, 64)), wspec((1, 64)),
            wspec((4, 256, 48)), wspec((1, 48)),
        ],
        out_specs=pl.BlockSpec((1, 56, 56, 48), lambda i: (i, 0, 0, 0)),
        out_shape=jax.ShapeDtypeStruct((n, 56, 56, 48), f32),
        scratch_shapes=[pltpu.VMEM((58, 56, 128), f32),
                        pltpu.VMEM((58, 56, 128), f32),
                        pltpu.VMEM((58, 56, 128), f32),
                        pltpu.VMEM((58, 56, 256), f32),
                        pltpu.VMEM((58, 56, 256), f32),
                        pltpu.VMEM((3136, 128), f32)],
    )(xa, xb, w0, p['b0'][None, :],
      rw1[0], r[0]['b1'][None, :], rw2[0], r[0]['b2'][None, :],
      rw1[1], r[1]['b1'][None, :], rw2[1], r[1]['b2'][None, :],
      ctw, p['ct1_b'][None, :], ct2w, ct2b[None, :])
    dec = (z2.reshape(n, 56, 56, 4, 4, 3)
             .transpose(0, 5, 1, 3, 2, 4).reshape(n, 3, 224, 224))
    return dec


def kernel(input, params):
    x = input.transpose(0, 2, 3, 1)
    enc_b = _encoder_s4(x, params['enc_b'])
    enc_t = _encoder_s2(enc_b, params['enc_t'])
    quant_t, diff_t = _quantize([enc_t], params['pre_t_w'], params['pre_t_b'],
                                params['embed_t'], 28)
    dec_t = _decoder_s2(quant_t, params['dec_t'])
    quant_b, diff_b = _quantize([dec_t, enc_b], params['pre_b_w'],
                                params['pre_b_b'], params['embed_b'], 56)
    diff = (diff_t + diff_b)[None]
    upsample_t = _conv_t(quant_t, params['post_t_w'], params['post_t_b'], 2, 1)
    dec = _decoder_s4_pallas(upsample_t, quant_b, params['dec_b'])
    return dec, diff.mean() * BETA


# speedup vs baseline: 1.1361x; 1.1361x over previous
import jax
import jax.numpy as jnp
from jax import lax
from jax.experimental import pallas as pl
from jax.experimental.pallas import tpu as pltpu

BETA = 0.25
ROW_BLOCK = 3136


def _vq_body(x, e_ref, et_ref, q_ref, part_ref):
    e = e_ref[:]
    e2 = jnp.sum(e * e, axis=0, keepdims=True)
    score = 2.0 * jnp.dot(x, e, preferred_element_type=jnp.float32) - e2
    ind = jnp.argmax(score, axis=1)
    onehot = (lax.broadcasted_iota(jnp.int32, score.shape, 1)
              == ind[:, None]).astype(jnp.float32)
    q = jnp.dot(onehot, et_ref[:], preferred_element_type=jnp.float32)
    q_ref[:] = q
    d = q - x
    part_ref[:] = jnp.full((1, 1, 128), jnp.sum(d * d), jnp.float32)


def _vq_block(x_ref, e_ref, et_ref, q_ref, part_ref):
    _vq_body(x_ref[:], e_ref, et_ref, q_ref, part_ref)


def _quantize(inp, embed):
    d = embed.shape[0]
    k = embed.shape[1]
    flat = inp.reshape(-1, d)
    n = flat.shape[0]
    grid = n // ROW_BLOCK
    q, parts = pl.pallas_call(
        _vq_block,
        grid=(grid,),
        in_specs=[
            pl.BlockSpec((ROW_BLOCK, d), lambda i: (i, 0)),
            pl.BlockSpec((d, k), lambda i: (0, 0)),
            pl.BlockSpec((k, d), lambda i: (0, 0)),
        ],
        out_specs=[
            pl.BlockSpec((ROW_BLOCK, d), lambda i: (i, 0)),
            pl.BlockSpec((1, 1, 128), lambda i: (i, 0, 0)),
        ],
        out_shape=[
            jax.ShapeDtypeStruct((n, d), jnp.float32),
            jax.ShapeDtypeStruct((grid, 1, 128), jnp.float32),
        ],
    )(flat, embed, embed.T)
    diff = jnp.sum(parts[:, 0, 0]) / (n * d)
    return q.reshape(inp.shape), diff


def _conv(x, w, b, stride=1, pad=0):
    y = lax.conv_general_dilated(x, w.transpose(2, 3, 1, 0), (stride, stride),
                                 [(pad, pad), (pad, pad)],
                                 dimension_numbers=('NHWC', 'HWIO', 'NHWC'),
                                 preferred_element_type=jnp.float32)
    return y + b[None, None, None, :]


def _conv_t(x, w, b, stride=2, pad=1):
    wt = jnp.flip(w, (2, 3)).transpose(2, 3, 0, 1)
    kk = w.shape[2]
    p = kk - 1 - pad
    y = lax.conv_general_dilated(x, wt, (1, 1), [(p, p), (p, p)],
                                 lhs_dilation=(stride, stride),
                                 dimension_numbers=('NHWC', 'HWIO', 'NHWC'),
                                 preferred_element_type=jnp.float32)
    return y + b[None, None, None, :]


def _res_block(x, p):
    o = jax.nn.relu(x)
    o = _conv(o, p['w1'], p['b1'], 1, 1)
    o = jax.nn.relu(o)
    o = _conv(o, p['w2'], p['b2'], 1, 0)
    return o + x


def _encoder_s4(x, p):
    x = jax.nn.relu(_conv(x, p['w0'], p['b0'], 2, 1))
    x = jax.nn.relu(_conv(x, p['w1'], p['b1'], 2, 1))
    x = _conv(x, p['w2'], p['b2'], 1, 1)
    for rp in p['res']:
        x = _res_block(x, rp)
    return jax.nn.relu(x)


def _encoder_s2(x, p):
    x = jax.nn.relu(_conv(x, p['w0'], p['b0'], 2, 1))
    x = _conv(x, p['w1'], p['b1'], 1, 1)
    for rp in p['res']:
        x = _res_block(x, rp)
    return jax.nn.relu(x)


def _decoder_s2(x, p):
    x = _conv(x, p['w0'], p['b0'], 1, 1)
    for rp in p['res']:
        x = _res_block(x, rp)
    x = jax.nn.relu(x)
    return _conv_t(x, p['ct_w'], p['ct_b'], 2, 1)


def _dec_stage1(xa_ref, xb_ref, w0_ref, b0_ref,
                r1w1_ref, r1b1_ref, r1w2_ref, r1b2_ref,
                r2w1_ref, r2b1_ref, r2w2_ref, r2b2_ref,
                ctw_ref, ctb_ref, ct2w_ref, ct2b_ref,
                z_ref, s0, s1, s2, sc0, sc1, y_s):
    f32 = jnp.float32
    H = 56
    M = H * H
    S = (s0, s1, s2)
    SC = (sc0, sc1)

    @pl.when(pl.program_id(0) == 0)
    def _zero():
        s0[:] = jnp.zeros_like(s0)
        s1[:] = jnp.zeros_like(s1)
        s2[:] = jnp.zeros_like(s2)
        sc0[:] = jnp.zeros_like(sc0)
        sc1[:] = jnp.zeros_like(sc1)

    def put(a):
        s0[1:57, 1:56, :] = a[:, 0:55, :]
        s1[1:57, 0:56, :] = a
        s2[1:57, 0:55, :] = a[:, 1:56, :]

    def conv9(w_ref, bias):
        acc = jnp.broadcast_to(bias, (M, w_ref.shape[2]))
        for dy in range(3):
            for dx in range(3):
                lhs = S[dx][dy:dy + H, :, :].reshape(M, 128)
                acc = acc + jnp.dot(lhs, w_ref[dy * 3 + dx],
                                    preferred_element_type=f32)
        return acc

    put(jnp.concatenate([xa_ref[0], xb_ref[0]], axis=2))

    y = conv9(w0_ref, b0_ref[:])
    y_s[:] = y
    put(jax.nn.relu(y).reshape(H, H, 128))

    for (w1_ref, b1_ref, w2_ref, b2_ref) in (
            (r1w1_ref, r1b1_ref, r1w2_ref, r1b2_ref),
            (r2w1_ref, r2b1_ref, r2w2_ref, r2b2_ref)):
        o = conv9(w1_ref, b1_ref[:])
        o = jax.nn.relu(o)
        o = jnp.dot(o, w2_ref[:], preferred_element_type=f32) + b2_ref[:]
        y = o + y_s[:]
        y_s[:] = y
        put(jax.nn.relu(y).reshape(H, H, 128))

    for r in range(2):
        for c in range(2):
            rc = r * 2 + c
            acc = jnp.broadcast_to(ctb_ref[:], (M, 64))
            for a in range(2):
                for b in range(2):
                    lhs = S[b + c][a + r:a + r + H, :, :].reshape(M, 128)
                    acc = acc + jnp.dot(lhs, ctw_ref[rc * 4 + a * 2 + b],
                                        preferred_element_type=f32)
            av = jax.nn.relu(acc).reshape(H, H, 64)
            if c == 0:
                SC[r][1:57, 0:56, 64:128] = av
                SC[r][1:57, 0:55, 192:256] = av[:, 1:56, :]
            else:
                SC[r][1:57, 1:56, 0:64] = av[:, 0:55, :]
                SC[r][1:57, 0:56, 128:192] = av

    out = jnp.broadcast_to(ct2b_ref[:], (M, 48))
    for g, (s, r1) in enumerate(((-1, 1), (0, 0), (0, 1), (1, 0))):
        lhs = SC[r1][1 + s:57 + s, :, :].reshape(M, 256)
        out = out + jnp.dot(lhs, ct2w_ref[g], preferred_element_type=f32)
    z_ref[0, :, :, :] = out.reshape(H, H, 48)


def _phase_w_t(w):
    ci, co = w.shape[0], w.shape[1]
    wt = jnp.flip(w, (2, 3)).transpose(2, 3, 0, 1)
    return (wt.reshape(2, 2, 2, 2, ci, co)
              .transpose(1, 3, 0, 2, 4, 5).reshape(16, ci, co))


def _ct2_group_w(w):
    wp = _phase_w_t(w)
    r6 = jnp.zeros((4, 4, 64, 4, 4, 3), jnp.float32)
    for u1 in range(2):
        for r2 in range(2):
            for a in range(2):
                gr = u1 + r2 + a
                pr = 2 * u1 + r2
                for v1 in range(2):
                    for c2 in range(2):
                        for b in range(2):
                            gc = v1 + c2 + b
                            pc = 2 * v1 + c2
                            wtap = wp[(r2 * 2 + c2) * 4 + a * 2 + b]
                            r6 = r6.at[gr, gc, :, pr, pc, :].set(wtap)
    return r6.reshape(4, 256, 48)


def _decoder_s4_pallas(xa, xb, p):
    n = xa.shape[0]
    f32 = jnp.float32
    w0 = p['w0'].transpose(2, 3, 1, 0).reshape(9, 128, 128)
    r = p['res']
    rw1 = [rp['w1'].transpose(2, 3, 1, 0).reshape(9, 128, 32) for rp in r]
    rw2 = [rp['w2'][:, :, 0, 0].T for rp in r]
    ctw = _phase_w_t(p['ct1_w'])
    ct2w = _ct2_group_w(p['ct2_w'])
    ct2b = jnp.tile(p['ct2_b'], 16)
    wspec = lambda s: pl.BlockSpec(s, lambda i: (0,) * len(s))
    z2 = pl.pallas_call(
        _dec_stage1,
        grid=(n,),
        in_specs=[
            pl.BlockSpec((1, 56, 56, 64), lambda i: (i, 0, 0, 0)),
            pl.BlockSpec((1, 56, 56, 64), lambda i: (i, 0, 0, 0)),
            wspec((9, 128, 128)), wspec((1, 128)),
            wspec((9, 128, 32)), wspec((1, 32)), wspec((32, 128)), wspec((1, 128)),
            wspec((9, 128, 32)), wspec((1, 32)), wspec((32, 128)), wspec((1, 128)),
            wspec((16, 128, 64)), wspec((1, 64)),
            wspec((4, 256, 48)), wspec((1, 48)),
        ],
        out_specs=pl.BlockSpec((1, 56, 56, 48), lambda i: (i, 0, 0, 0)),
        out_shape=jax.ShapeDtypeStruct((n, 56, 56, 48), f32),
        scratch_shapes=[pltpu.VMEM((58, 56, 128), f32),
                        pltpu.VMEM((58, 56, 128), f32),
                        pltpu.VMEM((58, 56, 128), f32),
                        pltpu.VMEM((58, 56, 256), f32),
                        pltpu.VMEM((58, 56, 256), f32),
                        pltpu.VMEM((3136, 128), f32)],
    )(xa, xb, w0, p['b0'][None, :],
      rw1[0], r[0]['b1'][None, :], rw2[0], r[0]['b2'][None, :],
      rw1[1], r[1]['b1'][None, :], rw2[1], r[1]['b2'][None, :],
      ctw, p['ct1_b'][None, :], ct2w, ct2b[None, :])
    dec = (z2.reshape(n, 56, 56, 4, 4, 3)
             .transpose(0, 5, 1, 3, 2, 4).reshape(n, 3, 224, 224))
    return dec


def kernel(input, params):
    x = input.transpose(0, 2, 3, 1)
    enc_b = _encoder_s4(x, params['enc_b'])
    enc_t = _encoder_s2(enc_b, params['enc_t'])
    qt_in = _conv(enc_t, params['pre_t_w'], params['pre_t_b'], 1, 0)
    quant_t, diff_t = _quantize(qt_in, params['embed_t'])
    dec_t = _decoder_s2(quant_t, params['dec_t'])
    cat_b = jnp.concatenate([dec_t, enc_b], axis=3)
    qb_in = _conv(cat_b, params['pre_b_w'], params['pre_b_b'], 1, 0)
    quant_b, diff_b = _quantize(qb_in, params['embed_b'])
    diff = (diff_t + diff_b)[None]
    upsample_t = _conv_t(quant_t, params['post_t_w'], params['post_t_b'], 2, 1)
    dec = _decoder_s4_pallas(upsample_t, quant_b, params['dec_b'])
    return dec, diff.mean() * BETA
